# Initial kernel scaffold; baseline (speedup 1.0000x reference)
import jax
import jax.numpy as jnp
from jax.experimental import pallas as pl


def kernel(X, A, W0, b0, W1, b1, W2, b2, Wl, bl):
    raise NotImplementedError("write your pallas kernel here")


# speedup vs baseline: 23.6666x; 23.6666x over previous
import functools

import jax
import jax.numpy as jnp
from jax import lax
from jax.experimental import pallas as pl
from jax.experimental.pallas import tpu as pltpu
from jax.experimental.pallas import tpu_sc as plsc

N = 10000
E = 320000
D = 128
H = 128
C = 40

NC = 2
NS = 16
NW = NC * NS

K = 128
STEPS = 80
CH = 10
NCHUNK = STEPS // CH
EW = K * STEPS
E_PAD = EW * NW
NDUMP = 16
NACC = N + NDUMP
ROWS_PER_TILE = 624
ROWS_TAIL = N - NS * ROWS_PER_TILE

_mesh = plsc.VectorSubcoreMesh(
    core_axis_name="c", subcore_axis_name="s", num_cores=NC, num_subcores=NS
)


@functools.partial(
    pl.kernel,
    out_type=jax.ShapeDtypeStruct((NC, N, H), jnp.float32),
    mesh=_mesh,
    scratch_types=[
        pltpu.VMEM((STEPS, K), jnp.int32),
        pltpu.VMEM((K, H), jnp.float32),
        pltpu.VMEM_SHARED((NACC, H), jnp.float32),
        pltpu.SemaphoreType.DMA,
    ],
)
def _deg_kernel(col_hbm, ones_hbm, zeros_hbm, out_hbm, col_v, ones_v, acc, sem):
    cid = lax.axis_index("c")
    sid = lax.axis_index("s")
    wid = sid * NC + cid
    pltpu.sync_copy(col_hbm.at[wid], col_v)
    pltpu.sync_copy(ones_hbm, ones_v)

    @pl.when(sid == 0)
    def _():
        pltpu.sync_copy(zeros_hbm, acc)

    plsc.subcore_barrier()

    def body(j, carry):
        pltpu.sync_copy(ones_v, acc.at[col_v.at[j]], add=True)
        return carry

    lax.fori_loop(0, STEPS, body, 0)
    plsc.subcore_barrier()
    base = sid * ROWS_PER_TILE
    pltpu.sync_copy(
        acc.at[pl.ds(base, ROWS_PER_TILE)],
        out_hbm.at[cid, pl.ds(base, ROWS_PER_TILE)],
    )

    @pl.when(sid == 0)
    def _():
        tb = NS * ROWS_PER_TILE
        pltpu.sync_copy(
            acc.at[pl.ds(tb, ROWS_TAIL)], out_hbm.at[cid, pl.ds(tb, ROWS_TAIL)]
        )


@functools.partial(
    pl.kernel,
    out_type=jax.ShapeDtypeStruct((NC, N, H), jnp.float32),
    mesh=_mesh,
    scratch_types=[
        pltpu.VMEM((2, CH, K), jnp.int32),
        pltpu.VMEM((2, CH, K), jnp.int32),
        pltpu.VMEM((K, H), jnp.float32),
        pltpu.VMEM((K, H), jnp.float32),
        pltpu.VMEM_SHARED((NACC, H), jnp.float32),
        pltpu.SemaphoreType.DMA,
        pltpu.SemaphoreType.DMA,
        pltpu.SemaphoreType.DMA,
        pltpu.SemaphoreType.DMA,
    ],
)
def _agg_kernel(xs_hbm, row_hbm, col_hbm, zeros_hbm, out_hbm,
                rv, cv, buf0, buf1, acc, gsem0, gsem1, isem0, isem1):
    cid = lax.axis_index("c")
    sid = lax.axis_index("s")
    wid = sid * NC + cid
    isems = (isem0, isem1)
    bufs = ((buf0, gsem0), (buf1, gsem1))

    pltpu.make_async_copy(row_hbm.at[wid, 0], rv.at[0], isem0).start()
    pltpu.make_async_copy(col_hbm.at[wid, 0], cv.at[0], isem0).start()
    pltpu.make_async_copy(row_hbm.at[wid, 1], rv.at[1], isem1).start()
    pltpu.make_async_copy(col_hbm.at[wid, 1], cv.at[1], isem1).start()

    @pl.when(sid == 0)
    def _():
        pltpu.sync_copy(zeros_hbm, acc)

    plsc.subcore_barrier()

    pltpu.make_async_copy(row_hbm.at[wid, 0], rv.at[0], isem0).wait()
    pltpu.make_async_copy(col_hbm.at[wid, 0], cv.at[0], isem0).wait()
    pltpu.make_async_copy(xs_hbm.at[rv.at[0, 0]], buf0, gsem0).start()
    pltpu.make_async_copy(xs_hbm.at[rv.at[0, 1]], buf1, gsem1).start()

    def chunk(c, p):
        rvp, cvp = rv.at[p], cv.at[p]
        q = 1 - p
        for s in range(CH):
            buf, gsem = bufs[s % 2]
            pltpu.make_async_copy(xs_hbm.at[rvp.at[s]], buf, gsem).wait()
            pltpu.sync_copy(buf, acc.at[cvp.at[s]], add=True)
            if s + 2 < CH:
                pltpu.make_async_copy(xs_hbm.at[rvp.at[s + 2]], buf, gsem).start()

        @pl.when(c + 1 < NCHUNK)
        def _():
            pltpu.make_async_copy(row_hbm.at[wid, c + 1], rv.at[q], isems[q]).wait()
            pltpu.make_async_copy(col_hbm.at[wid, c + 1], cv.at[q], isems[q]).wait()
            pltpu.make_async_copy(xs_hbm.at[rv.at[q, 0]], buf0, gsem0).start()
            pltpu.make_async_copy(xs_hbm.at[rv.at[q, 1]], buf1, gsem1).start()

        @pl.when(c + 2 < NCHUNK)
        def _():
            pltpu.make_async_copy(row_hbm.at[wid, c + 2], rvp, isems[p]).start()
            pltpu.make_async_copy(col_hbm.at[wid, c + 2], cvp, isems[p]).start()

    def outer(i, carry):
        c0 = 2 * i
        chunk(c0, 0)
        chunk(c0 + 1, 1)
        return carry

    lax.fori_loop(0, NCHUNK // 2, outer, 0)
    plsc.subcore_barrier()
    base = sid * ROWS_PER_TILE
    pltpu.sync_copy(
        acc.at[pl.ds(base, ROWS_PER_TILE)],
        out_hbm.at[cid, pl.ds(base, ROWS_PER_TILE)],
    )

    @pl.when(sid == 0)
    def _():
        tb = NS * ROWS_PER_TILE
        pltpu.sync_copy(
            acc.at[pl.ds(tb, ROWS_TAIL)], out_hbm.at[cid, pl.ds(tb, ROWS_TAIL)]
        )


BN = 2000
GRID = N // BN


def _dis_block(degp_ref):
    deg = degp_ref[0, :, 0:1] + degp_ref[1, :, 0:1] + 1.0
    return lax.rsqrt(deg)


def _norm_matmul(x, w_ref):
    w = w_ref[...]
    inv = lax.rsqrt(jnp.sum(w * w))
    h = lax.dot_general(x, w, (((1,), (1,)), ((), ())),
                        preferred_element_type=jnp.float32)
    return h * inv


def _tc_prep_body(x_ref, w_ref, degp_ref, hs_ref):
    dis = _dis_block(degp_ref)
    hs_ref[...] = _norm_matmul(x_ref[...], w_ref) * dis


def _tc_mid0_body(s_ref, hs_ref, degp_ref, b_ref, w_ref, x_ref, hsn_ref):
    dis = _dis_block(degp_ref)
    xo = dis * (s_ref[0] + s_ref[1] + hs_ref[...]) + b_ref[...]
    x_ref[...] = xo
    hsn_ref[...] = _norm_matmul(xo, w_ref) * dis


def _tc_mid_body(s_ref, hs_ref, degp_ref, b_ref, skip_ref, w_ref, x_ref, hsn_ref):
    dis = _dis_block(degp_ref)
    xo = jax.nn.relu(dis * (s_ref[0] + s_ref[1] + hs_ref[...]) + b_ref[...])
    xo = xo + skip_ref[...]
    x_ref[...] = xo
    hsn_ref[...] = _norm_matmul(xo, w_ref) * dis


def _tc_fin_body(s_ref, hs_ref, degp_ref, b_ref, skip_ref, wl_ref, bl_ref, y_ref):
    dis = _dis_block(degp_ref)
    xo = jax.nn.relu(dis * (s_ref[0] + s_ref[1] + hs_ref[...]) + b_ref[...])
    xo = xo + skip_ref[...]
    y = lax.dot_general(xo, wl_ref[...], (((1,), (1,)), ((), ())),
                        preferred_element_type=jnp.float32)
    y_ref[...] = y + bl_ref[...]


def _spec2d(width=H):
    return pl.BlockSpec((BN, width), lambda i: (i, 0))


def _spec3d(width=H):
    return pl.BlockSpec((2, BN, width), lambda i: (0, i, 0))


def _spec_w():
    return pl.BlockSpec((H, H), lambda i: (0, 0))


def _spec_b(width=H):
    return pl.BlockSpec((1, width), lambda i: (0, 0))


_out2d = jax.ShapeDtypeStruct((N, H), jnp.float32)

_tc_prep = pl.pallas_call(
    _tc_prep_body,
    grid=(GRID,),
    in_specs=[_spec2d(), _spec_w(), _spec3d()],
    out_specs=_spec2d(),
    out_shape=_out2d,
)

_tc_mid0 = pl.pallas_call(
    _tc_mid0_body,
    grid=(GRID,),
    in_specs=[_spec3d(), _spec2d(), _spec3d(), _spec_b(), _spec_w()],
    out_specs=(_spec2d(), _spec2d()),
    out_shape=(_out2d, _out2d),
)

_tc_mid = pl.pallas_call(
    _tc_mid_body,
    grid=(GRID,),
    in_specs=[_spec3d(), _spec2d(), _spec3d(), _spec_b(), _spec2d(), _spec_w()],
    out_specs=(_spec2d(), _spec2d()),
    out_shape=(_out2d, _out2d),
)

_tc_fin = pl.pallas_call(
    _tc_fin_body,
    grid=(GRID,),
    in_specs=[_spec3d(), _spec2d(), _spec3d(), _spec_b(), _spec2d(),
              _spec_w(), _spec_b()],
    out_specs=_spec2d(),
    out_shape=_out2d,
)


def kernel(X, A, W0, b0, W1, b1, W2, b2, Wl, bl):
    row = A[0]
    col = A[1]
    pad = E_PAD - E
    pr = (jnp.arange(pad, dtype=jnp.int32) * 797) % N
    pc = N + (jnp.arange(pad, dtype=jnp.int32) % NDUMP)
    rowp = jnp.concatenate([row, pr]).reshape(NW, NCHUNK, CH, K)
    colp_flat = jnp.concatenate([col, pc])
    colp = colp_flat.reshape(NW, NCHUNK, CH, K)
    colp3 = colp_flat.reshape(NW, STEPS, K)

    zeros_h = jnp.zeros((NACC, H), jnp.float32)
    ones_h = jnp.ones((K, H), jnp.float32)

    degp = _deg_kernel(colp3, ones_h, zeros_h)

    hs0 = _tc_prep(X, W0, degp)
    s0 = _agg_kernel(hs0, rowp, colp, zeros_h)
    x0, hs1 = _tc_mid0(s0, hs0, degp, b0.reshape(1, H), W1)
    s1 = _agg_kernel(hs1, rowp, colp, zeros_h)
    x1, hs2 = _tc_mid(s1, hs1, degp, b1.reshape(1, H), x0, W2)
    s2 = _agg_kernel(hs2, rowp, colp, zeros_h)

    wlp = jnp.zeros((H, H), jnp.float32).at[:C].set(Wl)
    blp = jnp.zeros((1, H), jnp.float32).at[0, :C].set(bl)
    y = _tc_fin(s2, hs2, degp, b2.reshape(1, H), x1, wlp, blp)
    return y[:, :C]


# speedup vs baseline: 23.7303x; 1.0027x over previous
import functools

import jax
import jax.numpy as jnp
from jax import lax
from jax.experimental import pallas as pl
from jax.experimental.pallas import tpu as pltpu
from jax.experimental.pallas import tpu_sc as plsc

N = 10000
E = 320000
D = 128
H = 128
C = 40

NC = 2
NS = 16
NW = NC * NS

K = 128
STEPS = 80
CH = 10
NCHUNK = STEPS // CH
EW = K * STEPS
E_PAD = EW * NW
NDUMP = 16
NACC = N + NDUMP
ROWS_PER_TILE = 624
ROWS_TAIL = N - NS * ROWS_PER_TILE

_mesh = plsc.VectorSubcoreMesh(
    core_axis_name="c", subcore_axis_name="s", num_cores=NC, num_subcores=NS
)


@functools.partial(
    pl.kernel,
    out_type=jax.ShapeDtypeStruct((NC, N, H), jnp.float32),
    mesh=_mesh,
    scratch_types=[
        pltpu.VMEM((STEPS, K), jnp.int32),
        pltpu.VMEM((K, H), jnp.float32),
        pltpu.VMEM_SHARED((NACC, H), jnp.float32),
        pltpu.SemaphoreType.DMA,
    ],
)
def _deg_kernel(col_hbm, ones_hbm, zeros_hbm, out_hbm, col_v, ones_v, acc, sem):
    cid = lax.axis_index("c")
    sid = lax.axis_index("s")
    wid = sid * NC + cid
    pltpu.sync_copy(col_hbm.at[wid], col_v)
    pltpu.sync_copy(ones_hbm, ones_v)

    @pl.when(sid == 0)
    def _():
        pltpu.sync_copy(zeros_hbm, acc)

    plsc.subcore_barrier()

    def body(j, carry):
        pltpu.sync_copy(ones_v, acc.at[col_v.at[j]], add=True)
        return carry

    lax.fori_loop(0, STEPS, body, 0)
    plsc.subcore_barrier()
    base = sid * ROWS_PER_TILE
    pltpu.sync_copy(
        acc.at[pl.ds(base, ROWS_PER_TILE)],
        out_hbm.at[cid, pl.ds(base, ROWS_PER_TILE)],
    )

    @pl.when(sid == 0)
    def _():
        tb = NS * ROWS_PER_TILE
        pltpu.sync_copy(
            acc.at[pl.ds(tb, ROWS_TAIL)], out_hbm.at[cid, pl.ds(tb, ROWS_TAIL)]
        )


@functools.partial(
    pl.kernel,
    out_type=jax.ShapeDtypeStruct((NC, N, H), jnp.float32),
    mesh=_mesh,
    scratch_types=[
        pltpu.VMEM((2, CH, K), jnp.int32),
        pltpu.VMEM((2, CH, K), jnp.int32),
        pltpu.VMEM((K, H), jnp.float32),
        pltpu.VMEM((K, H), jnp.float32),
        pltpu.VMEM_SHARED((NACC, H), jnp.float32),
        pltpu.SemaphoreType.DMA,
        pltpu.SemaphoreType.DMA,
        pltpu.SemaphoreType.DMA,
        pltpu.SemaphoreType.DMA,
    ],
)
def _agg_kernel(xs_hbm, row_hbm, col_hbm, zeros_hbm, out_hbm,
                rv, cv, buf0, buf1, acc, gsem0, gsem1, isem0, isem1):
    cid = lax.axis_index("c")
    sid = lax.axis_index("s")
    wid = sid * NC + cid
    isems = (isem0, isem1)
    bufs = ((buf0, gsem0), (buf1, gsem1))

    pltpu.make_async_copy(row_hbm.at[wid, 0], rv.at[0], isem0).start()
    pltpu.make_async_copy(col_hbm.at[wid, 0], cv.at[0], isem0).start()
    pltpu.make_async_copy(row_hbm.at[wid, 1], rv.at[1], isem1).start()
    pltpu.make_async_copy(col_hbm.at[wid, 1], cv.at[1], isem1).start()

    @pl.when(sid == 0)
    def _():
        pltpu.sync_copy(zeros_hbm, acc)

    plsc.subcore_barrier()

    pltpu.make_async_copy(row_hbm.at[wid, 0], rv.at[0], isem0).wait()
    pltpu.make_async_copy(col_hbm.at[wid, 0], cv.at[0], isem0).wait()
    pltpu.make_async_copy(xs_hbm.at[rv.at[0, 0]], buf0, gsem0).start()
    pltpu.make_async_copy(xs_hbm.at[rv.at[0, 1]], buf1, gsem1).start()

    def chunk(c, p):
        rvp, cvp = rv.at[p], cv.at[p]
        q = 1 - p
        for s in range(CH):
            buf, gsem = bufs[s % 2]
            pltpu.make_async_copy(xs_hbm.at[rvp.at[s]], buf, gsem).wait()
            pltpu.sync_copy(buf, acc.at[cvp.at[s]], add=True)
            if s + 2 < CH:
                pltpu.make_async_copy(xs_hbm.at[rvp.at[s + 2]], buf, gsem).start()

        @pl.when(c + 1 < NCHUNK)
        def _():
            pltpu.make_async_copy(row_hbm.at[wid, c + 1], rv.at[q], isems[q]).wait()
            pltpu.make_async_copy(col_hbm.at[wid, c + 1], cv.at[q], isems[q]).wait()
            pltpu.make_async_copy(xs_hbm.at[rv.at[q, 0]], buf0, gsem0).start()
            pltpu.make_async_copy(xs_hbm.at[rv.at[q, 1]], buf1, gsem1).start()

        @pl.when(c + 2 < NCHUNK)
        def _():
            pltpu.make_async_copy(row_hbm.at[wid, c + 2], rvp, isems[p]).start()
            pltpu.make_async_copy(col_hbm.at[wid, c + 2], cvp, isems[p]).start()

    def outer(i, carry):
        c0 = 2 * i
        chunk(c0, 0)
        chunk(c0 + 1, 1)
        return carry

    lax.fori_loop(0, NCHUNK // 2, outer, 0)
    plsc.subcore_barrier()
    base = sid * ROWS_PER_TILE
    pltpu.sync_copy(
        acc.at[pl.ds(base, ROWS_PER_TILE)],
        out_hbm.at[cid, pl.ds(base, ROWS_PER_TILE)],
    )

    @pl.when(sid == 0)
    def _():
        tb = NS * ROWS_PER_TILE
        pltpu.sync_copy(
            acc.at[pl.ds(tb, ROWS_TAIL)], out_hbm.at[cid, pl.ds(tb, ROWS_TAIL)]
        )


BN = 2000
GRID = N // BN


def _dis_block(degp_ref):
    deg = degp_ref[0, :, 0:1] + degp_ref[1, :, 0:1] + 1.0
    return lax.rsqrt(deg)


def _norm_matmul(x, w_ref):
    w = w_ref[...]
    inv = lax.rsqrt(jnp.sum(w * w))
    h = lax.dot_general(x, w, (((1,), (1,)), ((), ())),
                        preferred_element_type=jnp.float32)
    return h * inv


def _tc_prep_body(x_ref, w_ref, degp_ref, hs_ref, dis_ref):
    dis = _dis_block(degp_ref)
    dis_ref[...] = jnp.broadcast_to(dis, dis_ref.shape)
    hs_ref[...] = _norm_matmul(x_ref[...], w_ref) * dis


def _tc_mid0_body(s_ref, hs_ref, dis_b_ref, b_ref, w_ref, x_ref, hsn_ref):
    dis = dis_b_ref[:, 0:1]
    xo = dis * (s_ref[0] + s_ref[1] + hs_ref[...]) + b_ref[...]
    x_ref[...] = xo
    hsn_ref[...] = _norm_matmul(xo, w_ref) * dis


def _tc_mid_body(s_ref, hs_ref, dis_b_ref, b_ref, skip_ref, w_ref, x_ref, hsn_ref):
    dis = dis_b_ref[:, 0:1]
    xo = jax.nn.relu(dis * (s_ref[0] + s_ref[1] + hs_ref[...]) + b_ref[...])
    xo = xo + skip_ref[...]
    x_ref[...] = xo
    hsn_ref[...] = _norm_matmul(xo, w_ref) * dis


def _tc_fin_body(s_ref, hs_ref, dis_b_ref, b_ref, skip_ref, wl_ref, bl_ref, y_ref):
    dis = dis_b_ref[:, 0:1]
    xo = jax.nn.relu(dis * (s_ref[0] + s_ref[1] + hs_ref[...]) + b_ref[...])
    xo = xo + skip_ref[...]
    y = lax.dot_general(xo, wl_ref[...], (((1,), (1,)), ((), ())),
                        preferred_element_type=jnp.float32)
    y_ref[...] = y + bl_ref[...]


def _spec2d(width=H):
    return pl.BlockSpec((BN, width), lambda i: (i, 0))


def _spec3d(width=H):
    return pl.BlockSpec((2, BN, width), lambda i: (0, i, 0))


def _spec_w():
    return pl.BlockSpec((H, H), lambda i: (0, 0))


def _spec_b(width=H):
    return pl.BlockSpec((1, width), lambda i: (0, 0))


_out2d = jax.ShapeDtypeStruct((N, H), jnp.float32)

_tc_prep = pl.pallas_call(
    _tc_prep_body,
    grid=(GRID,),
    in_specs=[_spec2d(), _spec_w(), _spec3d()],
    out_specs=(_spec2d(), _spec2d()),
    out_shape=(_out2d, _out2d),
)

_tc_mid0 = pl.pallas_call(
    _tc_mid0_body,
    grid=(GRID,),
    in_specs=[_spec3d(), _spec2d(), _spec2d(), _spec_b(), _spec_w()],
    out_specs=(_spec2d(), _spec2d()),
    out_shape=(_out2d, _out2d),
)

_tc_mid = pl.pallas_call(
    _tc_mid_body,
    grid=(GRID,),
    in_specs=[_spec3d(), _spec2d(), _spec2d(), _spec_b(), _spec2d(), _spec_w()],
    out_specs=(_spec2d(), _spec2d()),
    out_shape=(_out2d, _out2d),
)

_tc_fin = pl.pallas_call(
    _tc_fin_body,
    grid=(GRID,),
    in_specs=[_spec3d(), _spec2d(), _spec2d(), _spec_b(), _spec2d(),
              _spec_w(), _spec_b()],
    out_specs=_spec2d(),
    out_shape=_out2d,
)


def kernel(X, A, W0, b0, W1, b1, W2, b2, Wl, bl):
    row = A[0]
    col = A[1]
    pad = E_PAD - E
    pr = (jnp.arange(pad, dtype=jnp.int32) * 797) % N
    pc = N + (jnp.arange(pad, dtype=jnp.int32) % NDUMP)
    rowp = jnp.concatenate([row, pr]).reshape(NW, NCHUNK, CH, K)
    colp_flat = jnp.concatenate([col, pc])
    colp = colp_flat.reshape(NW, NCHUNK, CH, K)
    colp3 = colp_flat.reshape(NW, STEPS, K)

    zeros_h = jnp.zeros((NACC, H), jnp.float32)
    ones_h = jnp.ones((K, H), jnp.float32)

    degp = _deg_kernel(colp3, ones_h, zeros_h)

    hs0, dis_b = _tc_prep(X, W0, degp)
    s0 = _agg_kernel(hs0, rowp, colp, zeros_h)
    x0, hs1 = _tc_mid0(s0, hs0, dis_b, b0.reshape(1, H), W1)
    s1 = _agg_kernel(hs1, rowp, colp, zeros_h)
    x1, hs2 = _tc_mid(s1, hs1, dis_b, b1.reshape(1, H), x0, W2)
    s2 = _agg_kernel(hs2, rowp, colp, zeros_h)

    wlp = jnp.zeros((H, H), jnp.float32).at[:C].set(Wl)
    blp = jnp.zeros((1, H), jnp.float32).at[0, :C].set(bl)
    y = _tc_fin(s2, hs2, dis_b, b2.reshape(1, H), x1, wlp, blp)
    return y[:, :C]


# speedup vs baseline: 23.7748x; 1.0019x over previous
import functools

import jax
import jax.numpy as jnp
from jax import lax
from jax.experimental import pallas as pl
from jax.experimental.pallas import tpu as pltpu
from jax.experimental.pallas import tpu_sc as plsc

N = 10000
E = 320000
D = 128
H = 128
C = 40

NC = 2
NS = 16
NW = NC * NS

K = 128
STEPS = 80
CH = 10
NCHUNK = STEPS // CH
EW = K * STEPS
E_PAD = EW * NW
NDUMP = 16
NACC = N + NDUMP
ROWS_PER_TILE = 624
ROWS_TAIL = N - NS * ROWS_PER_TILE

_mesh = plsc.VectorSubcoreMesh(
    core_axis_name="c", subcore_axis_name="s", num_cores=NC, num_subcores=NS
)


@functools.partial(
    pl.kernel,
    out_type=jax.ShapeDtypeStruct((NC, N, H), jnp.float32),
    mesh=_mesh,
    scratch_types=[
        pltpu.VMEM((STEPS, K), jnp.int32),
        pltpu.VMEM((K, H), jnp.float32),
        pltpu.VMEM_SHARED((NACC, H), jnp.float32),
        pltpu.SemaphoreType.DMA,
    ],
)
def _deg_kernel(col_hbm, ones_hbm, zeros_hbm, out_hbm, col_v, ones_v, acc, sem):
    cid = lax.axis_index("c")
    sid = lax.axis_index("s")
    wid = sid * NC + cid
    pltpu.sync_copy(col_hbm.at[wid], col_v)
    pltpu.sync_copy(ones_hbm, ones_v)

    @pl.when(sid == 0)
    def _():
        pltpu.sync_copy(zeros_hbm, acc)

    plsc.subcore_barrier()

    UNROLL = 4

    def body(i, carry):
        descs = [
            pltpu.async_copy(ones_v, acc.at[col_v.at[i * UNROLL + b]], sem, add=True)
            for b in range(UNROLL)
        ]
        for d in descs:
            d.wait()
        return carry

    lax.fori_loop(0, STEPS // UNROLL, body, 0)
    plsc.subcore_barrier()
    base = sid * ROWS_PER_TILE
    pltpu.sync_copy(
        acc.at[pl.ds(base, ROWS_PER_TILE)],
        out_hbm.at[cid, pl.ds(base, ROWS_PER_TILE)],
    )

    @pl.when(sid == 0)
    def _():
        tb = NS * ROWS_PER_TILE
        pltpu.sync_copy(
            acc.at[pl.ds(tb, ROWS_TAIL)], out_hbm.at[cid, pl.ds(tb, ROWS_TAIL)]
        )


@functools.partial(
    pl.kernel,
    out_type=jax.ShapeDtypeStruct((NC, N, H), jnp.float32),
    mesh=_mesh,
    scratch_types=[
        pltpu.VMEM((2, CH, K), jnp.int32),
        pltpu.VMEM((2, CH, K), jnp.int32),
        pltpu.VMEM((K, H), jnp.float32),
        pltpu.VMEM((K, H), jnp.float32),
        pltpu.VMEM_SHARED((NACC, H), jnp.float32),
        pltpu.SemaphoreType.DMA,
        pltpu.SemaphoreType.DMA,
        pltpu.SemaphoreType.DMA,
        pltpu.SemaphoreType.DMA,
    ],
)
def _agg_kernel(xs_hbm, row_hbm, col_hbm, zeros_hbm, out_hbm,
                rv, cv, buf0, buf1, acc, gsem0, gsem1, isem0, isem1):
    cid = lax.axis_index("c")
    sid = lax.axis_index("s")
    wid = sid * NC + cid
    isems = (isem0, isem1)
    bufs = ((buf0, gsem0), (buf1, gsem1))

    pltpu.make_async_copy(row_hbm.at[wid, 0], rv.at[0], isem0).start()
    pltpu.make_async_copy(col_hbm.at[wid, 0], cv.at[0], isem0).start()
    pltpu.make_async_copy(row_hbm.at[wid, 1], rv.at[1], isem1).start()
    pltpu.make_async_copy(col_hbm.at[wid, 1], cv.at[1], isem1).start()

    @pl.when(sid == 0)
    def _():
        pltpu.sync_copy(zeros_hbm, acc)

    plsc.subcore_barrier()

    pltpu.make_async_copy(row_hbm.at[wid, 0], rv.at[0], isem0).wait()
    pltpu.make_async_copy(col_hbm.at[wid, 0], cv.at[0], isem0).wait()
    pltpu.make_async_copy(xs_hbm.at[rv.at[0, 0]], buf0, gsem0).start()
    pltpu.make_async_copy(xs_hbm.at[rv.at[0, 1]], buf1, gsem1).start()

    def chunk(c, p):
        rvp, cvp = rv.at[p], cv.at[p]
        q = 1 - p
        for s in range(CH):
            buf, gsem = bufs[s % 2]
            pltpu.make_async_copy(xs_hbm.at[rvp.at[s]], buf, gsem).wait()
            pltpu.sync_copy(buf, acc.at[cvp.at[s]], add=True)
            if s + 2 < CH:
                pltpu.make_async_copy(xs_hbm.at[rvp.at[s + 2]], buf, gsem).start()

        @pl.when(c + 1 < NCHUNK)
        def _():
            pltpu.make_async_copy(row_hbm.at[wid, c + 1], rv.at[q], isems[q]).wait()
            pltpu.make_async_copy(col_hbm.at[wid, c + 1], cv.at[q], isems[q]).wait()
            pltpu.make_async_copy(xs_hbm.at[rv.at[q, 0]], buf0, gsem0).start()
            pltpu.make_async_copy(xs_hbm.at[rv.at[q, 1]], buf1, gsem1).start()

        @pl.when(c + 2 < NCHUNK)
        def _():
            pltpu.make_async_copy(row_hbm.at[wid, c + 2], rvp, isems[p]).start()
            pltpu.make_async_copy(col_hbm.at[wid, c + 2], cvp, isems[p]).start()

    def outer(i, carry):
        c0 = 2 * i
        chunk(c0, 0)
        chunk(c0 + 1, 1)
        return carry

    lax.fori_loop(0, NCHUNK // 2, outer, 0)
    plsc.subcore_barrier()
    base = sid * ROWS_PER_TILE
    pltpu.sync_copy(
        acc.at[pl.ds(base, ROWS_PER_TILE)],
        out_hbm.at[cid, pl.ds(base, ROWS_PER_TILE)],
    )

    @pl.when(sid == 0)
    def _():
        tb = NS * ROWS_PER_TILE
        pltpu.sync_copy(
            acc.at[pl.ds(tb, ROWS_TAIL)], out_hbm.at[cid, pl.ds(tb, ROWS_TAIL)]
        )


BN = 2000
GRID = N // BN


def _dis_block(degp_ref):
    deg = degp_ref[0, :, 0:1] + degp_ref[1, :, 0:1] + 1.0
    return lax.rsqrt(deg)


def _norm_matmul(x, w_ref):
    w = w_ref[...]
    inv = lax.rsqrt(jnp.sum(w * w))
    h = lax.dot_general(x, w, (((1,), (1,)), ((), ())),
                        preferred_element_type=jnp.float32)
    return h * inv


def _tc_mm_body(x_ref, w_ref, h_ref):
    h_ref[...] = _norm_matmul(x_ref[...], w_ref)


def _tc_scale_body(h_ref, degp_ref, hs_ref, dis_ref):
    dis = _dis_block(degp_ref)
    dis_ref[...] = jnp.broadcast_to(dis, dis_ref.shape)
    hs_ref[...] = h_ref[...] * dis


def _tc_mid0_body(s_ref, hs_ref, dis_b_ref, b_ref, w_ref, x_ref, hsn_ref):
    dis = dis_b_ref[:, 0:1]
    xo = dis * (s_ref[0] + s_ref[1] + hs_ref[...]) + b_ref[...]
    x_ref[...] = xo
    hsn_ref[...] = _norm_matmul(xo, w_ref) * dis


def _tc_mid_body(s_ref, hs_ref, dis_b_ref, b_ref, skip_ref, w_ref, x_ref, hsn_ref):
    dis = dis_b_ref[:, 0:1]
    xo = jax.nn.relu(dis * (s_ref[0] + s_ref[1] + hs_ref[...]) + b_ref[...])
    xo = xo + skip_ref[...]
    x_ref[...] = xo
    hsn_ref[...] = _norm_matmul(xo, w_ref) * dis


def _tc_fin_body(s_ref, hs_ref, dis_b_ref, b_ref, skip_ref, wl_ref, bl_ref, y_ref):
    dis = dis_b_ref[:, 0:1]
    xo = jax.nn.relu(dis * (s_ref[0] + s_ref[1] + hs_ref[...]) + b_ref[...])
    xo = xo + skip_ref[...]
    y = lax.dot_general(xo, wl_ref[...], (((1,), (1,)), ((), ())),
                        preferred_element_type=jnp.float32)
    y_ref[...] = y + bl_ref[...]


def _spec2d(width=H):
    return pl.BlockSpec((BN, width), lambda i: (i, 0))


def _spec3d(width=H):
    return pl.BlockSpec((2, BN, width), lambda i: (0, i, 0))


def _spec_w():
    return pl.BlockSpec((H, H), lambda i: (0, 0))


def _spec_b(width=H):
    return pl.BlockSpec((1, width), lambda i: (0, 0))


_out2d = jax.ShapeDtypeStruct((N, H), jnp.float32)

_tc_mm = pl.pallas_call(
    _tc_mm_body,
    grid=(GRID,),
    in_specs=[_spec2d(), _spec_w()],
    out_specs=_spec2d(),
    out_shape=_out2d,
)

_tc_scale = pl.pallas_call(
    _tc_scale_body,
    grid=(GRID,),
    in_specs=[_spec2d(), _spec3d()],
    out_specs=(_spec2d(), _spec2d()),
    out_shape=(_out2d, _out2d),
)

_tc_mid0 = pl.pallas_call(
    _tc_mid0_body,
    grid=(GRID,),
    in_specs=[_spec3d(), _spec2d(), _spec2d(), _spec_b(), _spec_w()],
    out_specs=(_spec2d(), _spec2d()),
    out_shape=(_out2d, _out2d),
)

_tc_mid = pl.pallas_call(
    _tc_mid_body,
    grid=(GRID,),
    in_specs=[_spec3d(), _spec2d(), _spec2d(), _spec_b(), _spec2d(), _spec_w()],
    out_specs=(_spec2d(), _spec2d()),
    out_shape=(_out2d, _out2d),
)

_tc_fin = pl.pallas_call(
    _tc_fin_body,
    grid=(GRID,),
    in_specs=[_spec3d(), _spec2d(), _spec2d(), _spec_b(), _spec2d(),
              _spec_w(), _spec_b()],
    out_specs=_spec2d(),
    out_shape=_out2d,
)


def kernel(X, A, W0, b0, W1, b1, W2, b2, Wl, bl):
    row = A[0]
    col = A[1]
    pad = E_PAD - E
    pr = (jnp.arange(pad, dtype=jnp.int32) * 797) % N
    pc = N + (jnp.arange(pad, dtype=jnp.int32) % NDUMP)
    rowp = jnp.concatenate([row, pr]).reshape(NW, NCHUNK, CH, K)
    colp_flat = jnp.concatenate([col, pc])
    colp = colp_flat.reshape(NW, NCHUNK, CH, K)
    colp3 = colp_flat.reshape(NW, STEPS, K)

    zeros_h = jnp.zeros((NACC, H), jnp.float32)
    ones_h = jnp.ones((K, H), jnp.float32)

    degp = _deg_kernel(colp3, ones_h, zeros_h)

    h0 = _tc_mm(X, W0)
    hs0, dis_b = _tc_scale(h0, degp)
    s0 = _agg_kernel(hs0, rowp, colp, zeros_h)
    x0, hs1 = _tc_mid0(s0, hs0, dis_b, b0.reshape(1, H), W1)
    s1 = _agg_kernel(hs1, rowp, colp, zeros_h)
    x1, hs2 = _tc_mid(s1, hs1, dis_b, b1.reshape(1, H), x0, W2)
    s2 = _agg_kernel(hs2, rowp, colp, zeros_h)

    wlp = jnp.zeros((H, H), jnp.float32).at[:C].set(Wl)
    blp = jnp.zeros((1, H), jnp.float32).at[0, :C].set(bl)
    y = _tc_fin(s2, hs2, dis_b, b2.reshape(1, H), x1, wlp, blp)
    return y[:, :C]


# speedup vs baseline: 23.7781x; 1.0001x over previous
import functools

import jax
import jax.numpy as jnp
from jax import lax
from jax.experimental import pallas as pl
from jax.experimental.pallas import tpu as pltpu
from jax.experimental.pallas import tpu_sc as plsc

N = 10000
E = 320000
D = 128
H = 128
C = 40

NC = 2
NS = 16
NW = NC * NS

K = 128
STEPS = 80
EW = K * STEPS
E_PAD = EW * NW

K2 = 80
STEPS2 = 128
CH = 16
NCHUNK = STEPS2 // CH
NDUMP = 16
NACC = N + NDUMP
ROWS_PER_TILE = 624
ROWS_TAIL = N - NS * ROWS_PER_TILE

_mesh = plsc.VectorSubcoreMesh(
    core_axis_name="c", subcore_axis_name="s", num_cores=NC, num_subcores=NS
)


@functools.partial(
    pl.kernel,
    out_type=jax.ShapeDtypeStruct((NC, N, H), jnp.float32),
    mesh=_mesh,
    scratch_types=[
        pltpu.VMEM((STEPS, K), jnp.int32),
        pltpu.VMEM((K, H), jnp.float32),
        pltpu.VMEM_SHARED((NACC, H), jnp.float32),
        pltpu.SemaphoreType.DMA,
    ],
)
def _deg_kernel(col_hbm, ones_hbm, zeros_hbm, out_hbm, col_v, ones_v, acc, sem):
    cid = lax.axis_index("c")
    sid = lax.axis_index("s")
    wid = sid * NC + cid
    pltpu.sync_copy(col_hbm.at[wid], col_v)
    pltpu.sync_copy(ones_hbm, ones_v)

    @pl.when(sid == 0)
    def _():
        pltpu.sync_copy(zeros_hbm, acc)

    plsc.subcore_barrier()

    UNROLL = 4

    def body(i, carry):
        descs = [
            pltpu.async_copy(ones_v, acc.at[col_v.at[i * UNROLL + b]], sem, add=True)
            for b in range(UNROLL)
        ]
        for d in descs:
            d.wait()
        return carry

    lax.fori_loop(0, STEPS // UNROLL, body, 0)
    plsc.subcore_barrier()
    base = sid * ROWS_PER_TILE
    pltpu.sync_copy(
        acc.at[pl.ds(base, ROWS_PER_TILE)],
        out_hbm.at[cid, pl.ds(base, ROWS_PER_TILE)],
    )

    @pl.when(sid == 0)
    def _():
        tb = NS * ROWS_PER_TILE
        pltpu.sync_copy(
            acc.at[pl.ds(tb, ROWS_TAIL)], out_hbm.at[cid, pl.ds(tb, ROWS_TAIL)]
        )


@functools.partial(
    pl.kernel,
    out_type=jax.ShapeDtypeStruct((NC, N, H), jnp.float32),
    mesh=_mesh,
    scratch_types=[
        pltpu.VMEM((2, CH, K2), jnp.int32),
        pltpu.VMEM((2, CH, K2), jnp.int32),
        pltpu.VMEM((K2, H), jnp.float32),
        pltpu.VMEM((K2, H), jnp.float32),
        pltpu.VMEM((K2, H), jnp.float32),
        pltpu.VMEM((K2, H), jnp.float32),
        pltpu.SemaphoreType.DMA,
        pltpu.SemaphoreType.DMA,
        pltpu.SemaphoreType.DMA,
        pltpu.SemaphoreType.DMA,
        pltpu.SemaphoreType.DMA,
        pltpu.SemaphoreType.DMA,
        pltpu.SemaphoreType.DMA,
        pltpu.SemaphoreType.DMA,
        pltpu.SemaphoreType.DMA,
        pltpu.SemaphoreType.DMA,
        pltpu.VMEM_SHARED((NACC, H), jnp.float32),
    ],
)
def _agg_kernel(xs_hbm, row_hbm, col_hbm, zeros_hbm, out_hbm,
                rv, cv, b0, b1, b2, b3,
                g0, g1, g2, g3, s0, s1, s2, s3, isem0, isem1, acc):
    cid = lax.axis_index("c")
    sid = lax.axis_index("s")
    wid = sid * NC + cid
    isems = (isem0, isem1)
    bufs = (b0, b1, b2, b3)
    gsems = (g0, g1, g2, g3)
    ssems = (s0, s1, s2, s3)

    def start_g(rvp, s, a):
        pltpu.make_async_copy(xs_hbm.at[rvp.at[s]], bufs[a], gsems[a]).start()

    def wait_g(rvp, s, a):
        pltpu.make_async_copy(xs_hbm.at[rvp.at[s]], bufs[a], gsems[a]).wait()

    def start_s(cvp, s, a):
        pltpu.make_async_copy(bufs[a], acc.at[cvp.at[s]], ssems[a]).start(add=True)

    def wait_s(cvp, s, a):
        pltpu.make_async_copy(bufs[a], acc.at[cvp.at[s]], ssems[a]).wait()

    pltpu.make_async_copy(row_hbm.at[wid, 0], rv.at[0], isem0).start()
    pltpu.make_async_copy(col_hbm.at[wid, 0], cv.at[0], isem0).start()
    pltpu.make_async_copy(row_hbm.at[wid, 1], rv.at[1], isem1).start()
    pltpu.make_async_copy(col_hbm.at[wid, 1], cv.at[1], isem1).start()

    @pl.when(sid == 0)
    def _():
        pltpu.sync_copy(zeros_hbm, acc)

    plsc.subcore_barrier()

    pltpu.make_async_copy(row_hbm.at[wid, 0], rv.at[0], isem0).wait()
    pltpu.make_async_copy(col_hbm.at[wid, 0], cv.at[0], isem0).wait()
    start_g(rv.at[0], 0, 0)
    start_g(rv.at[0], 1, 1)

    def chunk(c, p):
        rvp, cvp = rv.at[p], cv.at[p]
        q = 1 - p
        for s in range(CH):
            a = s % 4
            wait_g(rvp, s, a)
            start_s(cvp, s, a)
            if s + 2 < CH:
                a2 = (s + 2) % 4
                if s >= 2:
                    wait_s(cvp, s - 2, a2)
                start_g(rvp, s + 2, a2)
        wait_s(cvp, CH - 4, 0)
        wait_s(cvp, CH - 3, 1)

        @pl.when(c + 1 < NCHUNK)
        def _():
            pltpu.make_async_copy(row_hbm.at[wid, c + 1], rv.at[q], isems[q]).wait()
            pltpu.make_async_copy(col_hbm.at[wid, c + 1], cv.at[q], isems[q]).wait()
            start_g(rv.at[q], 0, 0)
            start_g(rv.at[q], 1, 1)

        wait_s(cvp, CH - 2, 2)
        wait_s(cvp, CH - 1, 3)

        @pl.when(c + 2 < NCHUNK)
        def _():
            pltpu.make_async_copy(row_hbm.at[wid, c + 2], rvp, isems[p]).start()
            pltpu.make_async_copy(col_hbm.at[wid, c + 2], cvp, isems[p]).start()

    def outer(i, carry):
        c0 = 2 * i
        chunk(c0, 0)
        chunk(c0 + 1, 1)
        return carry

    lax.fori_loop(0, NCHUNK // 2, outer, 0)
    plsc.subcore_barrier()
    base = sid * ROWS_PER_TILE
    pltpu.sync_copy(
        acc.at[pl.ds(base, ROWS_PER_TILE)],
        out_hbm.at[cid, pl.ds(base, ROWS_PER_TILE)],
    )

    @pl.when(sid == 0)
    def _():
        tb = NS * ROWS_PER_TILE
        pltpu.sync_copy(
            acc.at[pl.ds(tb, ROWS_TAIL)], out_hbm.at[cid, pl.ds(tb, ROWS_TAIL)]
        )


BN = 2000
GRID = N // BN


def _dis_block(degp_ref):
    deg = degp_ref[0, :, 0:1] + degp_ref[1, :, 0:1] + 1.0
    return lax.rsqrt(deg)


def _norm_matmul(x, w_ref):
    w = w_ref[...]
    inv = lax.rsqrt(jnp.sum(w * w))
    h = lax.dot_general(x, w, (((1,), (1,)), ((), ())),
                        preferred_element_type=jnp.float32)
    return h * inv


def _tc_mm_body(x_ref, w_ref, h_ref):
    h_ref[...] = _norm_matmul(x_ref[...], w_ref)


def _tc_scale_body(h_ref, degp_ref, hs_ref, dis_ref):
    dis = _dis_block(degp_ref)
    dis_ref[...] = jnp.broadcast_to(dis, dis_ref.shape)
    hs_ref[...] = h_ref[...] * dis


def _tc_mid0_body(s_ref, hs_ref, dis_b_ref, b_ref, w_ref, x_ref, hsn_ref):
    dis = dis_b_ref[:, 0:1]
    xo = dis * (s_ref[0] + s_ref[1] + hs_ref[...]) + b_ref[...]
    x_ref[...] = xo
    hsn_ref[...] = _norm_matmul(xo, w_ref) * dis


def _tc_mid_body(s_ref, hs_ref, dis_b_ref, b_ref, skip_ref, w_ref, x_ref, hsn_ref):
    dis = dis_b_ref[:, 0:1]
    xo = jax.nn.relu(dis * (s_ref[0] + s_ref[1] + hs_ref[...]) + b_ref[...])
    xo = xo + skip_ref[...]
    x_ref[...] = xo
    hsn_ref[...] = _norm_matmul(xo, w_ref) * dis


def _tc_fin_body(s_ref, hs_ref, dis_b_ref, b_ref, skip_ref, wl_ref, bl_ref, y_ref):
    dis = dis_b_ref[:, 0:1]
    xo = jax.nn.relu(dis * (s_ref[0] + s_ref[1] + hs_ref[...]) + b_ref[...])
    xo = xo + skip_ref[...]
    y = lax.dot_general(xo, wl_ref[...], (((1,), (1,)), ((), ())),
                        preferred_element_type=jnp.float32)
    y_ref[...] = y + bl_ref[...]


def _spec2d(width=H):
    return pl.BlockSpec((BN, width), lambda i: (i, 0))


def _spec3d(width=H):
    return pl.BlockSpec((2, BN, width), lambda i: (0, i, 0))


def _spec_w():
    return pl.BlockSpec((H, H), lambda i: (0, 0))


def _spec_b(width=H):
    return pl.BlockSpec((1, width), lambda i: (0, 0))


_out2d = jax.ShapeDtypeStruct((N, H), jnp.float32)

_tc_mm = pl.pallas_call(
    _tc_mm_body,
    grid=(GRID,),
    in_specs=[_spec2d(), _spec_w()],
    out_specs=_spec2d(),
    out_shape=_out2d,
)

_tc_scale = pl.pallas_call(
    _tc_scale_body,
    grid=(GRID,),
    in_specs=[_spec2d(), _spec3d()],
    out_specs=(_spec2d(), _spec2d()),
    out_shape=(_out2d, _out2d),
)

_tc_mid0 = pl.pallas_call(
    _tc_mid0_body,
    grid=(GRID,),
    in_specs=[_spec3d(), _spec2d(), _spec2d(), _spec_b(), _spec_w()],
    out_specs=(_spec2d(), _spec2d()),
    out_shape=(_out2d, _out2d),
)

_tc_mid = pl.pallas_call(
    _tc_mid_body,
    grid=(GRID,),
    in_specs=[_spec3d(), _spec2d(), _spec2d(), _spec_b(), _spec2d(), _spec_w()],
    out_specs=(_spec2d(), _spec2d()),
    out_shape=(_out2d, _out2d),
)

_tc_fin = pl.pallas_call(
    _tc_fin_body,
    grid=(GRID,),
    in_specs=[_spec3d(), _spec2d(), _spec2d(), _spec_b(), _spec2d(),
              _spec_w(), _spec_b()],
    out_specs=_spec2d(),
    out_shape=_out2d,
)


def kernel(X, A, W0, b0, W1, b1, W2, b2, Wl, bl):
    row = A[0]
    col = A[1]
    pad = E_PAD - E
    pr = (jnp.arange(pad, dtype=jnp.int32) * 797) % N
    pc = N + (jnp.arange(pad, dtype=jnp.int32) % NDUMP)
    rowp = jnp.concatenate([row, pr]).reshape(NW, NCHUNK, CH, K2)
    colp_flat = jnp.concatenate([col, pc])
    colp = colp_flat.reshape(NW, NCHUNK, CH, K2)
    colp3 = colp_flat.reshape(NW, STEPS, K)

    zeros_h = jnp.zeros((NACC, H), jnp.float32)
    ones_h = jnp.ones((K, H), jnp.float32)

    degp = _deg_kernel(colp3, ones_h, zeros_h)

    h0 = _tc_mm(X, W0)
    hs0, dis_b = _tc_scale(h0, degp)
    s0 = _agg_kernel(hs0, rowp, colp, zeros_h)
    x0, hs1 = _tc_mid0(s0, hs0, dis_b, b0.reshape(1, H), W1)
    s1 = _agg_kernel(hs1, rowp, colp, zeros_h)
    x1, hs2 = _tc_mid(s1, hs1, dis_b, b1.reshape(1, H), x0, W2)
    s2 = _agg_kernel(hs2, rowp, colp, zeros_h)

    wlp = jnp.zeros((H, H), jnp.float32).at[:C].set(Wl)
    blp = jnp.zeros((1, H), jnp.float32).at[0, :C].set(bl)
    y = _tc_fin(s2, hs2, dis_b, b2.reshape(1, H), x1, wlp, blp)
    return y[:, :C]
